# baseline (speedup 1.0000x reference)
import jax
import jax.numpy as jnp
from jax.experimental import pallas as pl
from jax.experimental.pallas import tpu as pltpu

ROIS = 90
RP = 96
H = 128
L = 3
NEG_SLOPE = 0.2
GPB = 4
NB = 64
NSTEPS = NB // GPB + 1

_R_LB1, _R_L2W, _R_LB2, _R_W0, _R_W1, _R_W2 = 15, 16, 18, 24, 32, 160


def _leaky(v):
    return jnp.maximum(v, NEG_SLOPE * v)


def _body(x_ref, ea_ref, eaf_ref, vb_ref, lw1_hbm_ref,
          out_ref, mean_ref, xs_ref, lw1_ref, sem):
    g = pl.program_id(0)

    @pl.when(g == 0)
    def _():
        pltpu.make_async_copy(lw1_hbm_ref, lw1_ref, sem).start()
        total = jnp.sum(eaf_ref[...])
        mean_ref[...] = jnp.full(
            mean_ref.shape,
            total / float(eaf_ref.shape[0] * eaf_ref.shape[1]), jnp.float32)

    @pl.when(g < NSTEPS - 1)
    def _():
        mean_ea = mean_ref[0, 0]
        req = jax.lax.broadcasted_iota(jnp.int32, (ROIS, ROIS), 0)
        leq = jax.lax.broadcasted_iota(jnp.int32, (ROIS, ROIS), 1)
        diag = jnp.where(req == leq, 1.0, 0.0)
        dnT = (((1,), (1,)), ((), ()))
        dnA = (((0,), (0,)), ((), ()))

        eas = [ea_ref[gi] for gi in range(GPB)]
        hs = None
        for l in range(L):
            a2 = vb_ref[5 * l:5 * l + 2, :]
            b = vb_ref[5 * l + 4:5 * l + 5, :]
            c = jnp.sum(vb_ref[5 * l + 3:5 * l + 4, :] *
                        vb_ref[5 * l + 2:5 * l + 3, :])

            if l == 0:
                w0 = vb_ref[_R_W0:_R_W0 + 3, :]
                h_all = jnp.dot(x_ref[...], w0)
                hs = [h_all[gi * ROIS:(gi + 1) * ROIS] for gi in range(GPB)]
            else:
                w = vb_ref[_R_W1:_R_W1 + H, :] if l == 1 else \
                    vb_ref[_R_W2:_R_W2 + H, :]
                hs = [jnp.dot(h, w) for h in hs]
            cols = [jax.lax.dot_general(h, a2, dnT) for h in hs]
            rows = [jax.lax.dot_general(a2, h, dnT) for h in hs]
            coefs = []
            for gi in range(GPB):
                asc = cols[gi][:, 0:1]
                asr = rows[gi][0:1, :]
                adr = rows[gi][1:2, :]
                mt = _leaky(asc + adr + c * eas[gi])
                sl = _leaky(asr + adr + c * mean_ea)
                amax = jnp.maximum(jnp.max(mt, axis=0, keepdims=True), sl)
                p = jnp.exp(mt - amax)
                es = jnp.exp(sl - amax)
                den = jnp.sum(p, axis=0, keepdims=True) + es + 1e-16
                coefs.append((p + diag * es) / den)
            hs = [jnp.maximum(
                jax.lax.dot_general(coefs[gi], hs[gi], dnA) + b, 0.0)
                for gi in range(GPB)]
            for gi in range(GPB):
                xs_ref[g * GPB + gi, :ROIS, H * l:H * (l + 1)] = hs[gi]

    @pl.when(g == NSTEPS - 1)
    def _():
        pltpu.make_async_copy(lw1_hbm_ref, lw1_ref, sem).wait()
        parts = [jnp.zeros((NB, 64), jnp.float32) for _ in range(6)]
        for p in range(ROIS):
            parts[p % 6] += jnp.dot(xs_ref[:, p, :], lw1_ref[p])
        z1 = parts[0] + parts[1] + parts[2] + parts[3] + parts[4] + parts[5]
        z1 = jnp.maximum(z1 + vb_ref[_R_LB1:_R_LB1 + 1, 0:64], 0.0)
        l2w = vb_ref[_R_L2W:_R_L2W + 2, 0:64]
        z2 = jax.lax.dot_general(z1, l2w, (((1,), (1,)), ((), ())))
        z2 = z2 + vb_ref[_R_LB2:_R_LB2 + 1, 0:2]
        m = jnp.max(z2, axis=1, keepdims=True)
        ssum = jnp.sum(jnp.exp(z2 - m), axis=1, keepdims=True)
        out_ref[...] = z2 - (m + jnp.log(ssum))


def kernel(x, edge_index, batch, edge_attr, params):
    del edge_index, batch
    e = edge_attr.shape[0]
    ea3 = edge_attr.reshape(NB, ROIS, ROIS)
    eaf = edge_attr.reshape(e // 128, 128)

    z64 = jnp.zeros((64,), jnp.float32)
    pieces = []
    for l in range(L):
        pieces += [params[f"as{l}"], params[f"ad{l}"], params[f"ae{l}"],
                   params[f"We{l}"].reshape(-1), params[f"b{l}"]]
    pieces += [params["lin1_b"], z64,
               params["lin2_w"][:, 0], z64,
               params["lin2_w"][:, 1], z64,
               params["lin2_b"], jnp.zeros((126,), jnp.float32),
               jnp.zeros((5 * 128,), jnp.float32),
               params["W0"].reshape(-1),
               jnp.zeros((5 * 128,), jnp.float32),
               params["W1"].reshape(-1),
               params["W2"].reshape(-1)]
    vb = jnp.concatenate(pieces).reshape(288, 128)

    last = NSTEPS - 2
    cspec = lambda shape: pl.BlockSpec(shape, lambda g: (0,) * len(shape))
    return pl.pallas_call(
        _body,
        grid=(NSTEPS,),
        in_specs=[
            pl.BlockSpec((GPB * ROIS, 3), lambda g: (jnp.minimum(g, last), 0)),
            pl.BlockSpec((GPB, ROIS, ROIS),
                         lambda g: (jnp.minimum(g, last), 0, 0)),
            cspec((e // 128, 128)),
            cspec((288, 128)),
            pl.BlockSpec(memory_space=pltpu.MemorySpace.HBM),
        ],
        out_specs=pl.BlockSpec((NB, 2), lambda g: (0, 0)),
        out_shape=jax.ShapeDtypeStruct((NB, 2), jnp.float32),
        scratch_shapes=[
            pltpu.VMEM((8, 128), jnp.float32),
            pltpu.VMEM((NB, RP, L * H), jnp.float32),
            pltpu.VMEM((ROIS, L * H, 64), jnp.float32),
            pltpu.SemaphoreType.DMA,
        ],
    )(x, ea3, eaf, vb, params["lin1_w"].reshape(ROIS, L * H, 64))


# speedup vs baseline: 1.3750x; 1.3750x over previous
import jax
import jax.numpy as jnp
from jax.experimental import pallas as pl
from jax.experimental.pallas import tpu as pltpu

ROIS = 90
RP = 96
H = 128
L = 3
NEG_SLOPE = 0.2
GPB = 16
NB = 64
NSTEPS = NB // GPB + 1

_R_LB1, _R_L2W, _R_LB2, _R_W0, _R_W1, _R_W2 = 15, 16, 18, 24, 32, 160


def _leaky(v):
    return jnp.maximum(v, NEG_SLOPE * v)


def _body(x_ref, ea_ref, eaf_ref, vb_ref, lw1_hbm_ref,
          out_ref, mean_ref, xs_ref, lw1_ref, sem):
    g = pl.program_id(0)

    @pl.when(g == 0)
    def _():
        pltpu.make_async_copy(lw1_hbm_ref, lw1_ref, sem).start()
        total = jnp.sum(eaf_ref[...])
        mean_ref[...] = jnp.full(
            mean_ref.shape,
            total / float(eaf_ref.shape[0] * eaf_ref.shape[1]), jnp.float32)

    @pl.when(g < NSTEPS - 1)
    def _():
        mean_ea = mean_ref[0, 0]
        req = jax.lax.broadcasted_iota(jnp.int32, (ROIS, ROIS), 0)
        leq = jax.lax.broadcasted_iota(jnp.int32, (ROIS, ROIS), 1)
        diag = jnp.where(req == leq, 1.0, 0.0)
        dnT = (((1,), (1,)), ((), ()))
        dnA = (((0,), (0,)), ((), ()))

        eas = [ea_ref[gi] for gi in range(GPB)]
        hs = None
        for l in range(L):
            a2 = vb_ref[5 * l:5 * l + 2, :]
            b = vb_ref[5 * l + 4:5 * l + 5, :]
            c = jnp.sum(vb_ref[5 * l + 3:5 * l + 4, :] *
                        vb_ref[5 * l + 2:5 * l + 3, :])

            if l == 0:
                w0 = vb_ref[_R_W0:_R_W0 + 3, :]
                h_all = jnp.dot(x_ref[...], w0)
                hs = [h_all[gi * ROIS:(gi + 1) * ROIS] for gi in range(GPB)]
            else:
                w = vb_ref[_R_W1:_R_W1 + H, :] if l == 1 else \
                    vb_ref[_R_W2:_R_W2 + H, :]
                hs = [jnp.dot(h, w) for h in hs]
            cols = [jax.lax.dot_general(h, a2, dnT) for h in hs]
            rows = [jax.lax.dot_general(a2, h, dnT) for h in hs]
            coefs = []
            for gi in range(GPB):
                asc = cols[gi][:, 0:1]
                asr = rows[gi][0:1, :]
                adr = rows[gi][1:2, :]
                mt = _leaky(asc + adr + c * eas[gi])
                sl = _leaky(asr + adr + c * mean_ea)
                amax = jnp.maximum(jnp.max(mt, axis=0, keepdims=True), sl)
                p = jnp.exp(mt - amax)
                es = jnp.exp(sl - amax)
                den = jnp.sum(p, axis=0, keepdims=True) + es + 1e-16
                coefs.append((p + diag * es) / den)
            hs = [jnp.maximum(
                jax.lax.dot_general(coefs[gi], hs[gi], dnA) + b, 0.0)
                for gi in range(GPB)]
            for gi in range(GPB):
                xs_ref[g * GPB + gi, :ROIS, H * l:H * (l + 1)] = hs[gi]

    @pl.when(g == NSTEPS - 1)
    def _():
        pltpu.make_async_copy(lw1_hbm_ref, lw1_ref, sem).wait()
        parts = [jnp.zeros((NB, 64), jnp.float32) for _ in range(6)]
        for p in range(ROIS):
            parts[p % 6] += jnp.dot(xs_ref[:, p, :], lw1_ref[p])
        z1 = parts[0] + parts[1] + parts[2] + parts[3] + parts[4] + parts[5]
        z1 = jnp.maximum(z1 + vb_ref[_R_LB1:_R_LB1 + 1, 0:64], 0.0)
        l2w = vb_ref[_R_L2W:_R_L2W + 2, 0:64]
        z2 = jax.lax.dot_general(z1, l2w, (((1,), (1,)), ((), ())))
        z2 = z2 + vb_ref[_R_LB2:_R_LB2 + 1, 0:2]
        m = jnp.max(z2, axis=1, keepdims=True)
        ssum = jnp.sum(jnp.exp(z2 - m), axis=1, keepdims=True)
        out_ref[...] = z2 - (m + jnp.log(ssum))


def kernel(x, edge_index, batch, edge_attr, params):
    del edge_index, batch
    e = edge_attr.shape[0]
    ea3 = edge_attr.reshape(NB, ROIS, ROIS)
    eaf = edge_attr.reshape(e // 128, 128)

    z64 = jnp.zeros((64,), jnp.float32)
    pieces = []
    for l in range(L):
        pieces += [params[f"as{l}"], params[f"ad{l}"], params[f"ae{l}"],
                   params[f"We{l}"].reshape(-1), params[f"b{l}"]]
    pieces += [params["lin1_b"], z64,
               params["lin2_w"][:, 0], z64,
               params["lin2_w"][:, 1], z64,
               params["lin2_b"], jnp.zeros((126,), jnp.float32),
               jnp.zeros((5 * 128,), jnp.float32),
               params["W0"].reshape(-1),
               jnp.zeros((5 * 128,), jnp.float32),
               params["W1"].reshape(-1),
               params["W2"].reshape(-1)]
    vb = jnp.concatenate(pieces).reshape(288, 128)

    last = NSTEPS - 2
    cspec = lambda shape: pl.BlockSpec(shape, lambda g: (0,) * len(shape))
    return pl.pallas_call(
        _body,
        grid=(NSTEPS,),
        in_specs=[
            pl.BlockSpec((GPB * ROIS, 3), lambda g: (jnp.minimum(g, last), 0)),
            pl.BlockSpec((GPB, ROIS, ROIS),
                         lambda g: (jnp.minimum(g, last), 0, 0)),
            cspec((e // 128, 128)),
            cspec((288, 128)),
            pl.BlockSpec(memory_space=pltpu.MemorySpace.HBM),
        ],
        out_specs=pl.BlockSpec((NB, 2), lambda g: (0, 0)),
        out_shape=jax.ShapeDtypeStruct((NB, 2), jnp.float32),
        scratch_shapes=[
            pltpu.VMEM((8, 128), jnp.float32),
            pltpu.VMEM((NB, RP, L * H), jnp.float32),
            pltpu.VMEM((ROIS, L * H, 64), jnp.float32),
            pltpu.SemaphoreType.DMA,
        ],
    )(x, ea3, eaf, vb, params["lin1_w"].reshape(ROIS, L * H, 64))
